# Initial kernel scaffold; baseline (speedup 1.0000x reference)
import jax
import jax.numpy as jnp
from jax.experimental import pallas as pl


def kernel(x, edge_index, W1, b1, W2, b2, W3, b3, Wfc, bfc):
    raise NotImplementedError("write your pallas kernel here")


# speedup vs baseline: 5.2999x; 5.2999x over previous
import functools

import jax
import jax.numpy as jnp
from jax import lax
from jax.experimental import pallas as pl
from jax.experimental.pallas import tpu as pltpu
from jax.experimental.pallas import tpu_sc as plsc

B, N, E = 4, 10000, 160000
D_IN, D1, D2, D3, D_OUT = 128, 64, 128, 256, 512

NP = 10240
TRASH = N
EP = 163840
K = 128
RT = NP // 16
EPT = EP // 16
EPT2 = EP // 32
RB = 1024
NB = NP // RB

_MESH = dict(core_axis_name="c", subcore_axis_name="s")
_SC_PARAMS = pltpu.CompilerParams(use_tc_tiling_on_sc=False)


def _deg_call(dst_p):
    mesh = plsc.VectorSubcoreMesh(**_MESH)

    @functools.partial(
        pl.kernel,
        out_type=jax.ShapeDtypeStruct((2, B, NP), jnp.float32),
        mesh=mesh,
        compiler_params=_SC_PARAMS,
        scratch_types=[
            pltpu.VMEM_SHARED((NP,), jnp.float32),
            pltpu.VMEM((RT,), jnp.float32),
            pltpu.VMEM((K,), jnp.float32),
            pltpu.VMEM((K,), jnp.int32),
        ],
    )
    def k(dst_hbm, cnt_hbm, deg_sh, zbuf, ones_v, idx_v):
        c = lax.axis_index("c")
        s = lax.axis_index("s")

        @pl.loop(0, RT, step=16)
        def _(i):
            zbuf[pl.ds(i, 16)] = jnp.zeros((16,), jnp.float32)

        @pl.loop(0, K, step=16)
        def _(i):
            ones_v[pl.ds(i, 16)] = jnp.ones((16,), jnp.float32)

        @pl.loop(0, B)
        def _(b):
            pltpu.sync_copy(zbuf, deg_sh.at[pl.ds(s * RT, RT)])
            plsc.subcore_barrier()
            base = c * (EP // 2) + s * EPT2

            @pl.loop(0, EPT2 // K)
            def _(kk):
                pltpu.sync_copy(dst_hbm.at[b, pl.ds(base + kk * K, K)], idx_v)
                pltpu.sync_copy(ones_v, deg_sh.at[idx_v], add=True)

            plsc.subcore_barrier()
            pltpu.sync_copy(deg_sh.at[pl.ds(s * RT, RT)],
                            cnt_hbm.at[c, b, pl.ds(s * RT, RT)])

    return k(dst_p)


def _make_scatter(Dh):
    mesh = plsc.VectorSubcoreMesh(**_MESH)

    @functools.partial(
        pl.kernel,
        out_type=jax.ShapeDtypeStruct((B * 2 * NP, Dh), jnp.float32),
        mesh=mesh,
        compiler_params=_SC_PARAMS,
        scratch_types=[
            pltpu.VMEM_SHARED((NP, Dh), jnp.float32),
            pltpu.VMEM((K,), jnp.int32),
            pltpu.VMEM((K,), jnp.int32),
            pltpu.VMEM((K, Dh), jnp.float32),
        ],
    )
    def k(hs_hbm, src_hbm, dst_hbm, out_hbm, acc_sh, sidx, didx, rows):
        c = lax.axis_index("c")
        s = lax.axis_index("s")

        @pl.loop(0, B)
        def _(b):
            rowbase = (b * 2 + c) * NP
            pltpu.sync_copy(hs_hbm.at[pl.ds(rowbase + s * RT, RT)],
                            acc_sh.at[pl.ds(s * RT, RT)])
            plsc.subcore_barrier()

            @pl.loop(0, EPT // K)
            def _(kk):
                off = s * EPT + kk * K
                pltpu.sync_copy(src_hbm.at[b, pl.ds(off, K)], sidx)
                pltpu.sync_copy(dst_hbm.at[b, pl.ds(off, K)], didx)
                for j in range(K // 16):
                    sidx[pl.ds(j * 16, 16)] = sidx[pl.ds(j * 16, 16)] + rowbase
                pltpu.sync_copy(hs_hbm.at[sidx], rows)
                pltpu.sync_copy(rows, acc_sh.at[didx], add=True)

            plsc.subcore_barrier()
            pltpu.sync_copy(acc_sh.at[pl.ds(s * RT, RT)],
                            out_hbm.at[pl.ds(rowbase + s * RT, RT)])

    return k


_scatter32 = _make_scatter(32)
_scatter64 = _make_scatter(64)
_scatter128 = _make_scatter(128)


def _tc1_body(x_ref, w_ref, cnt_ref, hs_ref, dis_ref):
    deg = cnt_ref[0, 0] + cnt_ref[1, 0] + 1.0
    dis = lax.rsqrt(deg)
    h = jnp.dot(x_ref[0], w_ref[...], preferred_element_type=jnp.float32)
    hs = h * dis
    hs_ref[0, 0] = hs[:, :D1 // 2]
    hs_ref[0, 1] = hs[:, D1 // 2:]
    dis_ref[0] = dis


def _tc1_call(x_p, W1, cnt2):
    return pl.pallas_call(
        _tc1_body,
        grid=(B, NB),
        in_specs=[
            pl.BlockSpec((1, RB, D_IN), lambda b, n: (b, n, 0)),
            pl.BlockSpec((D_IN, D1), lambda b, n: (0, 0)),
            pl.BlockSpec((2, 1, RB, 1), lambda b, n: (0, b, n, 0)),
        ],
        out_specs=[
            pl.BlockSpec((1, 2, RB, D1 // 2), lambda b, n: (b, 0, n, 0)),
            pl.BlockSpec((1, RB, 1), lambda b, n: (b, n, 0)),
        ],
        out_shape=[
            jax.ShapeDtypeStruct((B, 2, NP, D1 // 2), jnp.float32),
            jax.ShapeDtypeStruct((B, NP, 1), jnp.float32),
        ],
    )(x_p, W1, cnt2)


def _tcmid_body(dh_in, dh_out, acc_ref, dis_ref, w_ref, b_ref, out_ref):
    dis = dis_ref[0]
    bias = b_ref[...]
    g0 = jnp.maximum(acc_ref[0, 0] * dis + bias[:dh_in][None, :], 0.0)
    g1 = jnp.maximum(acc_ref[0, 1] * dis + bias[dh_in:][None, :], 0.0)
    w = w_ref[...]
    h = (jnp.dot(g0, w[:dh_in], preferred_element_type=jnp.float32)
         + jnp.dot(g1, w[dh_in:], preferred_element_type=jnp.float32))
    hs = h * dis
    out_ref[0, 0] = hs[:, :dh_out]
    out_ref[0, 1] = hs[:, dh_out:]


def _tcmid_call(acc, dis, W, bias, dh_in, dh_out):
    return pl.pallas_call(
        functools.partial(_tcmid_body, dh_in, dh_out),
        grid=(B, NB),
        in_specs=[
            pl.BlockSpec((1, 2, RB, dh_in), lambda b, n: (b, 0, n, 0)),
            pl.BlockSpec((1, RB, 1), lambda b, n: (b, n, 0)),
            pl.BlockSpec((2 * dh_in, 2 * dh_out), lambda b, n: (0, 0)),
            pl.BlockSpec((2 * dh_in,), lambda b, n: (0,)),
        ],
        out_specs=pl.BlockSpec((1, 2, RB, dh_out), lambda b, n: (b, 0, n, 0)),
        out_shape=jax.ShapeDtypeStruct((B, 2, NP, dh_out), jnp.float32),
    )(acc, dis, W, bias)


def _tc4_body(acc_ref, dis_ref, b_ref, wfc_ref, bfc_ref, out_ref, pool_scr):
    n = pl.program_id(1)
    dis = dis_ref[0]
    bias = b_ref[...]
    g0 = jnp.maximum(acc_ref[0, 0] * dis + bias[:D3 // 2][None, :], 0.0)
    g1 = jnp.maximum(acc_ref[0, 1] * dis + bias[D3 // 2:][None, :], 0.0)
    rowid = lax.broadcasted_iota(jnp.int32, (RB, 1), 0) + n * RB
    m = (rowid < N).astype(jnp.float32)
    g0 = g0 * m
    g1 = g1 * m
    p = jnp.concatenate([jnp.sum(g0, axis=0), jnp.sum(g1, axis=0)])[None, :]

    @pl.when(n == 0)
    def _():
        pool_scr[...] = p

    @pl.when(n > 0)
    def _():
        pool_scr[...] = pool_scr[...] + p

    @pl.when(n == NB - 1)
    def _():
        pooled = pool_scr[...] * (1.0 / N)
        res = (jnp.dot(pooled, wfc_ref[...],
                       preferred_element_type=jnp.float32)
               + bfc_ref[...][None, :])
        out_ref[0] = jnp.broadcast_to(res, (8, D_OUT))


def _tc4_call(acc, dis, b3, Wfc, bfc):
    return pl.pallas_call(
        _tc4_body,
        grid=(B, NB),
        in_specs=[
            pl.BlockSpec((1, 2, RB, D3 // 2), lambda b, n: (b, 0, n, 0)),
            pl.BlockSpec((1, RB, 1), lambda b, n: (b, n, 0)),
            pl.BlockSpec((D3,), lambda b, n: (0,)),
            pl.BlockSpec((D3, D_OUT), lambda b, n: (0, 0)),
            pl.BlockSpec((D_OUT,), lambda b, n: (0,)),
        ],
        out_specs=pl.BlockSpec((1, 8, D_OUT), lambda b, n: (b, 0, 0)),
        out_shape=jax.ShapeDtypeStruct((B, 8, D_OUT), jnp.float32),
        scratch_shapes=[pltpu.VMEM((1, D3), jnp.float32)],
    )(acc, dis, b3, Wfc, bfc)


def kernel(x, edge_index, W1, b1, W2, b2, W3, b3, Wfc, bfc):
    src = edge_index[:, 0, :].astype(jnp.int32)
    dst = edge_index[:, 1, :].astype(jnp.int32)
    src_p = jnp.pad(src, ((0, 0), (0, EP - E)))
    dst_p = jnp.pad(dst, ((0, 0), (0, EP - E)), constant_values=TRASH)
    x_p = jnp.pad(x, ((0, 0), (0, NP - N), (0, 0)))

    cnt2 = _deg_call(dst_p).reshape(2, B, NP, 1)
    hs1, dis = _tc1_call(x_p, W1, cnt2)
    acc1 = _scatter32(hs1.reshape(B * 2 * NP, D1 // 2), src_p, dst_p)
    hs2 = _tcmid_call(acc1.reshape(B, 2, NP, D1 // 2), dis, W2, b1,
                      D1 // 2, D2 // 2)
    acc2 = _scatter64(hs2.reshape(B * 2 * NP, D2 // 2), src_p, dst_p)
    hs3 = _tcmid_call(acc2.reshape(B, 2, NP, D2 // 2), dis, W3, b2,
                      D2 // 2, D3 // 2)
    acc3 = _scatter128(hs3.reshape(B * 2 * NP, D3 // 2), src_p, dst_p)
    return _tc4_call(acc3.reshape(B, 2, NP, D3 // 2), dis, b3, Wfc, bfc)[:, 0, :]


# speedup vs baseline: 7.6428x; 1.4420x over previous
import functools

import jax
import jax.numpy as jnp
from jax import lax
from jax.experimental import pallas as pl
from jax.experimental.pallas import tpu as pltpu
from jax.experimental.pallas import tpu_sc as plsc

B, N, E = 4, 10000, 160000
D_IN, D1, D2, D3, D_OUT = 128, 64, 128, 256, 512

NP = 10240
TRASH = N
EP = 163840
K = 128
RT = NP // 16
EPT = EP // 16
EPT2 = EP // 32
RB = 1024
NB = NP // RB

_MESH = dict(core_axis_name="c", subcore_axis_name="s")
_SC_PARAMS = pltpu.CompilerParams(use_tc_tiling_on_sc=False)


G2 = 4
GK2 = G2 * K
NG2 = EPT2 // GK2


def _deg_call(dst_p):
    mesh = plsc.VectorSubcoreMesh(**_MESH)
    dstv = dst_p.reshape(B, 2, 16, NG2, G2, K)

    @functools.partial(
        pl.kernel,
        out_type=jax.ShapeDtypeStruct((2, B, NP), jnp.float32),
        mesh=mesh,
        compiler_params=_SC_PARAMS,
        scratch_types=[
            pltpu.VMEM_SHARED((NP,), jnp.float32),
            pltpu.VMEM((RT,), jnp.float32),
            pltpu.VMEM((K,), jnp.float32),
            pltpu.VMEM((G2, K), jnp.int32),
            pltpu.VMEM((G2, K), jnp.int32),
            pltpu.SemaphoreType.DMA, pltpu.SemaphoreType.DMA,
            pltpu.SemaphoreType.DMA, pltpu.SemaphoreType.DMA,
            pltpu.SemaphoreType.DMA,
        ],
    )
    def k(dst_hbm, cnt_hbm, deg_sh, zbuf, ones_v, didx0, didx1,
          isem0, isem1, ssem0, ssem1, wsem):
        c = lax.axis_index("c")
        s = lax.axis_index("s")
        didx = [didx0, didx1]
        isem = [isem0, isem1]
        ssem = [ssem0, ssem1]

        @pl.loop(0, RT, step=16)
        def _(i):
            zbuf[pl.ds(i, 16)] = jnp.zeros((16,), jnp.float32)

        @pl.loop(0, K, step=16)
        def _(i):
            ones_v[pl.ds(i, 16)] = jnp.ones((16,), jnp.float32)

        @pl.loop(0, B)
        def _(b):
            def load_idx(g, p):
                pltpu.async_copy(dst_hbm.at[b, c, s, g], didx[p], isem[p])

            def wait_idx(p):
                pltpu.make_async_copy(dst_hbm.at[b, c, s, 0], didx[p],
                                      isem[p]).wait()

            def start_scatters(p):
                for j in range(G2):
                    pltpu.async_copy(ones_v, deg_sh.at[didx[p].at[j]],
                                     ssem[p], add=True)

            def wait_scatters(p):
                for j in range(G2):
                    pltpu.make_async_copy(ones_v, deg_sh.at[didx[p].at[j]],
                                          ssem[p]).wait()

            init = pltpu.async_copy(zbuf, deg_sh.at[pl.ds(s * RT, RT)], wsem)
            load_idx(0, 0)
            load_idx(1, 1)
            init.wait()
            plsc.subcore_barrier()

            @pl.loop(0, (NG2 - 2) // 2)
            def _(t):
                g = 2 * t
                wait_idx(0)
                start_scatters(0)
                wait_idx(1)
                start_scatters(1)
                wait_scatters(0)
                load_idx(g + 2, 0)
                wait_scatters(1)
                load_idx(g + 3, 1)

            wait_idx(0)
            start_scatters(0)
            wait_idx(1)
            start_scatters(1)
            wait_scatters(0)
            wait_scatters(1)
            plsc.subcore_barrier()
            pltpu.sync_copy(deg_sh.at[pl.ds(s * RT, RT)],
                            cnt_hbm.at[c, b, pl.ds(s * RT, RT)])

    return k(dstv)


def _make_scatter(W, npass):
    GK = 32768 // W
    G = GK // K
    NG = EPT // GK
    NS = 2 * npass
    mesh = plsc.VectorSubcoreMesh(**_MESH)

    @functools.partial(
        pl.kernel,
        out_type=jax.ShapeDtypeStruct((B * NS * NP, W), jnp.float32),
        mesh=mesh,
        compiler_params=_SC_PARAMS,
        scratch_types=[
            pltpu.VMEM_SHARED((NP, W), jnp.float32),
            pltpu.VMEM((GK,), jnp.int32), pltpu.VMEM((GK,), jnp.int32),
            pltpu.VMEM((G, K), jnp.int32), pltpu.VMEM((G, K), jnp.int32),
            pltpu.VMEM((GK, W), jnp.float32),
            pltpu.VMEM((GK, W), jnp.float32),
            pltpu.SemaphoreType.DMA, pltpu.SemaphoreType.DMA,
            pltpu.SemaphoreType.DMA, pltpu.SemaphoreType.DMA,
            pltpu.SemaphoreType.DMA, pltpu.SemaphoreType.DMA,
            pltpu.SemaphoreType.DMA,
        ],
    )
    def k(hs_hbm, srcv, dstv, out_hbm, acc_sh,
          sidx0, sidx1, didx0, didx1, rows0, rows1,
          isem0, isem1, gsem0, gsem1, ssem0, ssem1, wsem):
        c = lax.axis_index("c")
        s = lax.axis_index("s")
        sidx = [sidx0, sidx1]
        didx = [didx0, didx1]
        rows = [rows0, rows1]
        isem = [isem0, isem1]
        gsem = [gsem0, gsem1]
        ssem = [ssem0, ssem1]

        @pl.loop(0, B * npass)
        def _(u):
            b = u // npass
            q = u - b * npass
            rowbase = ((b * 2 + c) * npass + q) * NP

            def load_idx(g, p):
                pltpu.async_copy(srcv.at[b, s, g], sidx[p], isem[p])
                pltpu.async_copy(dstv.at[b, s, g], didx[p], isem[p])

            def start_gathers(p):
                pltpu.make_async_copy(srcv.at[b, s, 0], sidx[p],
                                      isem[p]).wait()
                pltpu.make_async_copy(dstv.at[b, s, 0], didx[p],
                                      isem[p]).wait()

                @pl.loop(0, GK, step=16)
                def _(i):
                    sidx[p][pl.ds(i, 16)] = sidx[p][pl.ds(i, 16)] + rowbase

                for j in range(G):
                    pltpu.async_copy(hs_hbm.at[sidx[p].at[pl.ds(j * K, K)]],
                                     rows[p].at[pl.ds(j * K, K)], gsem[p])

            def wait_gathers(p):
                for j in range(G):
                    pltpu.make_async_copy(
                        hs_hbm.at[sidx[p].at[pl.ds(j * K, K)]],
                        rows[p].at[pl.ds(j * K, K)], gsem[p]).wait()

            def start_scatters(p):
                for j in range(G):
                    pltpu.async_copy(rows[p].at[pl.ds(j * K, K)],
                                     acc_sh.at[didx[p].at[j]], ssem[p],
                                     add=True)

            def wait_scatters(p):
                for j in range(G):
                    pltpu.make_async_copy(rows[p].at[pl.ds(j * K, K)],
                                          acc_sh.at[didx[p].at[j]],
                                          ssem[p]).wait()

            init = pltpu.async_copy(hs_hbm.at[pl.ds(rowbase + s * RT, RT)],
                                    acc_sh.at[pl.ds(s * RT, RT)], wsem)
            load_idx(0, 0)
            load_idx(1, 1)
            start_gathers(0)
            init.wait()
            plsc.subcore_barrier()

            @pl.loop(0, (NG - 2) // 2)
            def _(t):
                g = 2 * t
                wait_gathers(0)
                start_scatters(0)
                start_gathers(1)
                wait_scatters(0)
                load_idx(g + 2, 0)
                wait_gathers(1)
                start_scatters(1)
                start_gathers(0)
                wait_scatters(1)
                load_idx(g + 3, 1)

            wait_gathers(0)
            start_scatters(0)
            start_gathers(1)
            wait_scatters(0)
            wait_gathers(1)
            start_scatters(1)
            wait_scatters(1)
            plsc.subcore_barrier()
            pltpu.sync_copy(acc_sh.at[pl.ds(s * RT, RT)],
                            out_hbm.at[pl.ds(rowbase + s * RT, RT)])

    def call(hs_flat, src_p, dst_p):
        return k(hs_flat, src_p.reshape(B, 16, NG, GK),
                 dst_p.reshape(B, 16, NG, G, K))

    return call


_scatter1 = _make_scatter(D1 // 2, 1)
_scatter2 = _make_scatter(D2 // 2, 1)
_scatter3 = _make_scatter(D3 // 4, 2)


def _tc1_body(x_ref, w_ref, cnt_ref, hs_ref, dis_ref):
    deg = cnt_ref[0, 0] + cnt_ref[1, 0] + 1.0
    dis = lax.rsqrt(deg)
    h = jnp.dot(x_ref[0], w_ref[...], preferred_element_type=jnp.float32)
    hs = h * dis
    w_out = D1 // 2
    for i in range(2):
        hs_ref[0, i] = hs[:, i * w_out:(i + 1) * w_out]
    dis_ref[0] = dis


def _tc1_call(x_p, W1, cnt2):
    return pl.pallas_call(
        _tc1_body,
        grid=(B, NB),
        in_specs=[
            pl.BlockSpec((1, RB, D_IN), lambda b, n: (b, n, 0)),
            pl.BlockSpec((D_IN, D1), lambda b, n: (0, 0)),
            pl.BlockSpec((2, 1, RB, 1), lambda b, n: (0, b, n, 0)),
        ],
        out_specs=[
            pl.BlockSpec((1, 2, RB, D1 // 2), lambda b, n: (b, 0, n, 0)),
            pl.BlockSpec((1, RB, 1), lambda b, n: (b, n, 0)),
        ],
        out_shape=[
            jax.ShapeDtypeStruct((B, 2, NP, D1 // 2), jnp.float32),
            jax.ShapeDtypeStruct((B, NP, 1), jnp.float32),
        ],
    )(x_p, W1, cnt2)


def _tcmid_body(w_in, s_in, w_out, s_out, acc_ref, dis_ref, w_ref, b_ref,
                out_ref):
    dis = dis_ref[0]
    bias = b_ref[...]
    w = w_ref[...]
    h = None
    for i in range(s_in):
        g = jnp.maximum(acc_ref[0, i] * dis
                        + bias[i * w_in:(i + 1) * w_in][None, :], 0.0)
        hi = jnp.dot(g, w[i * w_in:(i + 1) * w_in],
                     preferred_element_type=jnp.float32)
        h = hi if h is None else h + hi
    hs = h * dis
    for i in range(s_out):
        out_ref[0, i] = hs[:, i * w_out:(i + 1) * w_out]


def _tcmid_call(acc, dis, W, bias, w_in, s_in, w_out, s_out):
    d_in, d_out = w_in * s_in, w_out * s_out
    return pl.pallas_call(
        functools.partial(_tcmid_body, w_in, s_in, w_out, s_out),
        grid=(B, NB),
        in_specs=[
            pl.BlockSpec((1, s_in, RB, w_in), lambda b, n: (b, 0, n, 0)),
            pl.BlockSpec((1, RB, 1), lambda b, n: (b, n, 0)),
            pl.BlockSpec((d_in, d_out), lambda b, n: (0, 0)),
            pl.BlockSpec((d_in,), lambda b, n: (0,)),
        ],
        out_specs=pl.BlockSpec((1, s_out, RB, w_out),
                               lambda b, n: (b, 0, n, 0)),
        out_shape=jax.ShapeDtypeStruct((B, s_out, NP, w_out), jnp.float32),
    )(acc, dis, W, bias)


def _tc4_body(acc_ref, dis_ref, b_ref, wfc_ref, bfc_ref, out_ref, pool_scr):
    n = pl.program_id(1)
    dis = dis_ref[0]
    bias = b_ref[...]
    w_in = D3 // 4
    rowid = lax.broadcasted_iota(jnp.int32, (RB, 1), 0) + n * RB
    m = (rowid < N).astype(jnp.float32)
    parts = []
    for i in range(4):
        g = jnp.maximum(acc_ref[0, i] * dis
                        + bias[i * w_in:(i + 1) * w_in][None, :], 0.0)
        parts.append(jnp.sum(g * m, axis=0))
    p = jnp.concatenate(parts)[None, :]

    @pl.when(n == 0)
    def _():
        pool_scr[...] = p

    @pl.when(n > 0)
    def _():
        pool_scr[...] = pool_scr[...] + p

    @pl.when(n == NB - 1)
    def _():
        pooled = pool_scr[...] * (1.0 / N)
        res = (jnp.dot(pooled, wfc_ref[...],
                       preferred_element_type=jnp.float32)
               + bfc_ref[...][None, :])
        out_ref[0] = jnp.broadcast_to(res, (8, D_OUT))


def _tc4_call(acc, dis, b3, Wfc, bfc):
    return pl.pallas_call(
        _tc4_body,
        grid=(B, NB),
        in_specs=[
            pl.BlockSpec((1, 4, RB, D3 // 4), lambda b, n: (b, 0, n, 0)),
            pl.BlockSpec((1, RB, 1), lambda b, n: (b, n, 0)),
            pl.BlockSpec((D3,), lambda b, n: (0,)),
            pl.BlockSpec((D3, D_OUT), lambda b, n: (0, 0)),
            pl.BlockSpec((D_OUT,), lambda b, n: (0,)),
        ],
        out_specs=pl.BlockSpec((1, 8, D_OUT), lambda b, n: (b, 0, 0)),
        out_shape=jax.ShapeDtypeStruct((B, 8, D_OUT), jnp.float32),
        scratch_shapes=[pltpu.VMEM((1, D3), jnp.float32)],
    )(acc, dis, b3, Wfc, bfc)


def kernel(x, edge_index, W1, b1, W2, b2, W3, b3, Wfc, bfc):
    src = edge_index[:, 0, :].astype(jnp.int32)
    dst = edge_index[:, 1, :].astype(jnp.int32)
    src_p = jnp.pad(src, ((0, 0), (0, EP - E)))
    dst_p = jnp.pad(dst, ((0, 0), (0, EP - E)), constant_values=TRASH)
    x_p = jnp.pad(x, ((0, 0), (0, NP - N), (0, 0)))

    cnt2 = _deg_call(dst_p).reshape(2, B, NP, 1)
    hs1, dis = _tc1_call(x_p, W1, cnt2)
    acc1 = _scatter1(hs1.reshape(B * 2 * NP, D1 // 2), src_p, dst_p)
    hs2 = _tcmid_call(acc1.reshape(B, 2, NP, D1 // 2), dis, W2, b1,
                      D1 // 2, 2, D2 // 2, 2)
    acc2 = _scatter2(hs2.reshape(B * 2 * NP, D2 // 2), src_p, dst_p)
    hs3 = _tcmid_call(acc2.reshape(B, 2, NP, D2 // 2), dis, W3, b2,
                      D2 // 2, 2, D3 // 4, 4)
    acc3 = _scatter3(hs3.reshape(B * 4 * NP, D3 // 4), src_p, dst_p)
    return _tc4_call(acc3.reshape(B, 4, NP, D3 // 4), dis, b3, Wfc, bfc)[:, 0, :]


# speedup vs baseline: 12.2059x; 1.5971x over previous
import functools

import jax
import jax.numpy as jnp
from jax import lax
from jax.experimental import pallas as pl
from jax.experimental.pallas import tpu as pltpu
from jax.experimental.pallas import tpu_sc as plsc

B, N, E = 4, 10000, 160000
D_IN, D1, D2, D3, D_OUT = 128, 64, 128, 256, 512

NP = 10240
TRASH = N
EP = 163840
K = 128
RT = NP // 16
EPT = EP // 16
EPT2 = EP // 32
RB = 1024
NB = NP // RB

_MESH = dict(core_axis_name="c", subcore_axis_name="s")
_SC_PARAMS = pltpu.CompilerParams(use_tc_tiling_on_sc=False)


G2 = 4
GK2 = G2 * K
NG2 = EPT2 // GK2


def _deg_call(dst_p):
    mesh = plsc.VectorSubcoreMesh(**_MESH)
    dstv = dst_p.reshape(B, 2, 16, NG2, G2, K)

    @functools.partial(
        pl.kernel,
        out_type=jax.ShapeDtypeStruct((2, B, NP), jnp.float32),
        mesh=mesh,
        compiler_params=_SC_PARAMS,
        scratch_types=[
            pltpu.VMEM_SHARED((NP,), jnp.float32),
            pltpu.VMEM((RT,), jnp.float32),
            pltpu.VMEM((K,), jnp.float32),
            pltpu.VMEM((G2, K), jnp.int32),
            pltpu.VMEM((G2, K), jnp.int32),
            pltpu.SemaphoreType.DMA, pltpu.SemaphoreType.DMA,
            pltpu.SemaphoreType.DMA, pltpu.SemaphoreType.DMA,
            pltpu.SemaphoreType.DMA,
        ],
    )
    def k(dst_hbm, cnt_hbm, deg_sh, zbuf, ones_v, didx0, didx1,
          isem0, isem1, ssem0, ssem1, wsem):
        c = lax.axis_index("c")
        s = lax.axis_index("s")
        didx = [didx0, didx1]
        isem = [isem0, isem1]
        ssem = [ssem0, ssem1]

        @pl.loop(0, RT, step=16)
        def _(i):
            zbuf[pl.ds(i, 16)] = jnp.zeros((16,), jnp.float32)

        @pl.loop(0, K, step=16)
        def _(i):
            ones_v[pl.ds(i, 16)] = jnp.ones((16,), jnp.float32)

        @pl.loop(0, B)
        def _(b):
            def load_idx(g, p):
                pltpu.async_copy(dst_hbm.at[b, c, s, g], didx[p], isem[p])

            def wait_idx(p):
                pltpu.make_async_copy(dst_hbm.at[b, c, s, 0], didx[p],
                                      isem[p]).wait()

            def start_scatters(p):
                for j in range(G2):
                    pltpu.async_copy(ones_v, deg_sh.at[didx[p].at[j]],
                                     ssem[p], add=True)

            def wait_scatters(p):
                for j in range(G2):
                    pltpu.make_async_copy(ones_v, deg_sh.at[didx[p].at[j]],
                                          ssem[p]).wait()

            init = pltpu.async_copy(zbuf, deg_sh.at[pl.ds(s * RT, RT)], wsem)
            load_idx(0, 0)
            load_idx(1, 1)
            init.wait()
            plsc.subcore_barrier()

            @pl.loop(0, (NG2 - 2) // 2)
            def _(t):
                g = 2 * t
                wait_idx(0)
                start_scatters(0)
                wait_idx(1)
                start_scatters(1)
                wait_scatters(0)
                load_idx(g + 2, 0)
                wait_scatters(1)
                load_idx(g + 3, 1)

            wait_idx(0)
            start_scatters(0)
            wait_idx(1)
            start_scatters(1)
            wait_scatters(0)
            wait_scatters(1)
            plsc.subcore_barrier()
            pltpu.sync_copy(deg_sh.at[pl.ds(s * RT, RT)],
                            cnt_hbm.at[c, b, pl.ds(s * RT, RT)])

    return k(dstv)


def _make_scatter(W, npass):
    GK = 1024
    NG = EPT // GK
    NS = 2 * npass
    mesh = plsc.VectorSubcoreMesh(**_MESH)

    @functools.partial(
        pl.kernel,
        out_type=jax.ShapeDtypeStruct((B * NS, NP, W), jnp.bfloat16),
        mesh=mesh,
        compiler_params=_SC_PARAMS,
        scratch_types=[
            pltpu.VMEM_SHARED((NP, W), jnp.bfloat16),
            pltpu.VMEM((GK,), jnp.int32), pltpu.VMEM((GK,), jnp.int32),
            pltpu.VMEM((GK,), jnp.int32), pltpu.VMEM((GK,), jnp.int32),
            pltpu.VMEM((GK, W), jnp.bfloat16),
            pltpu.VMEM((GK, W), jnp.bfloat16),
            pltpu.SemaphoreType.DMA, pltpu.SemaphoreType.DMA,
            pltpu.SemaphoreType.DMA, pltpu.SemaphoreType.DMA,
            pltpu.SemaphoreType.DMA, pltpu.SemaphoreType.DMA,
            pltpu.SemaphoreType.DMA,
        ],
    )
    def k(hs_hbm, srcv, dstv, out_hbm, acc_sh,
          sidx0, sidx1, didx0, didx1, rows0, rows1,
          isem0, isem1, gsem0, gsem1, ssem0, ssem1, wsem):
        c = lax.axis_index("c")
        s = lax.axis_index("s")
        sidx = [sidx0, sidx1]
        didx = [didx0, didx1]
        rows = [rows0, rows1]
        isem = [isem0, isem1]
        gsem = [gsem0, gsem1]
        ssem = [ssem0, ssem1]

        @pl.loop(0, B * npass)
        def _(u):
            b = u // npass
            q = u - b * npass
            bcq = (b * 2 + c) * npass + q

            def load_idx(g, p):
                pltpu.async_copy(srcv.at[b, s, g], sidx[p], isem[p])
                pltpu.async_copy(dstv.at[b, s, g], didx[p], isem[p])

            def start_gathers(p):
                pltpu.make_async_copy(srcv.at[b, s, 0], sidx[p],
                                      isem[p]).wait()
                pltpu.make_async_copy(dstv.at[b, s, 0], didx[p],
                                      isem[p]).wait()
                pltpu.async_copy(hs_hbm.at[bcq].at[sidx[p]], rows[p],
                                 gsem[p])

            def wait_gathers(p):
                pltpu.make_async_copy(hs_hbm.at[bcq].at[sidx[p]], rows[p],
                                      gsem[p]).wait()

            def start_scatters(p):
                pltpu.async_copy(rows[p], acc_sh.at[didx[p]], ssem[p],
                                 add=True)

            def wait_scatters(p):
                pltpu.make_async_copy(rows[p], acc_sh.at[didx[p]],
                                      ssem[p]).wait()

            init = pltpu.async_copy(hs_hbm.at[bcq, pl.ds(s * RT, RT)],
                                    acc_sh.at[pl.ds(s * RT, RT)], wsem)
            load_idx(0, 0)
            load_idx(1, 1)
            start_gathers(0)
            init.wait()
            plsc.subcore_barrier()

            @pl.loop(0, (NG - 2) // 2)
            def _(t):
                g = 2 * t
                wait_gathers(0)
                start_scatters(0)
                start_gathers(1)
                wait_scatters(0)
                load_idx(g + 2, 0)
                wait_gathers(1)
                start_scatters(1)
                start_gathers(0)
                wait_scatters(1)
                load_idx(g + 3, 1)

            wait_gathers(0)
            start_scatters(0)
            start_gathers(1)
            wait_scatters(0)
            wait_gathers(1)
            start_scatters(1)
            wait_scatters(1)
            plsc.subcore_barrier()
            pltpu.sync_copy(acc_sh.at[pl.ds(s * RT, RT)],
                            out_hbm.at[bcq, pl.ds(s * RT, RT)])

    def call(hs_flat, src_p, dst_p):
        return k(hs_flat.reshape(B * NS, NP, W),
                 src_p.reshape(B, 16, NG, GK),
                 dst_p.reshape(B, 16, NG, GK))

    return call


_scatter1 = _make_scatter(D1 // 2, 1)
_scatter2 = _make_scatter(D2 // 2, 1)
_scatter3 = _make_scatter(D3 // 4, 2)


def _tc1_body(x_ref, w_ref, cnt_ref, hs_ref, dis_ref):
    deg = cnt_ref[0, 0] + cnt_ref[1, 0] + 1.0
    dis = lax.rsqrt(deg)
    h = jnp.dot(x_ref[0], w_ref[...], preferred_element_type=jnp.float32)
    hs = (h * dis).astype(jnp.bfloat16)
    w_out = D1 // 2
    for i in range(2):
        hs_ref[0, i] = hs[:, i * w_out:(i + 1) * w_out]
    dis_ref[0] = dis


def _tc1_call(x_p, W1, cnt2):
    return pl.pallas_call(
        _tc1_body,
        grid=(B, NB),
        in_specs=[
            pl.BlockSpec((1, RB, D_IN), lambda b, n: (b, n, 0)),
            pl.BlockSpec((D_IN, D1), lambda b, n: (0, 0)),
            pl.BlockSpec((2, 1, RB, 1), lambda b, n: (0, b, n, 0)),
        ],
        out_specs=[
            pl.BlockSpec((1, 2, RB, D1 // 2), lambda b, n: (b, 0, n, 0)),
            pl.BlockSpec((1, RB, 1), lambda b, n: (b, n, 0)),
        ],
        out_shape=[
            jax.ShapeDtypeStruct((B, 2, NP, D1 // 2), jnp.bfloat16),
            jax.ShapeDtypeStruct((B, NP, 1), jnp.float32),
        ],
    )(x_p, W1, cnt2)


def _tcmid_body(w_in, s_in, w_out, s_out, acc_ref, dis_ref, w_ref, b_ref,
                out_ref):
    dis = dis_ref[0]
    bias = b_ref[...]
    w = w_ref[...]
    h = None
    for i in range(s_in):
        g = jnp.maximum(acc_ref[0, i].astype(jnp.float32) * dis
                        + bias[i * w_in:(i + 1) * w_in][None, :], 0.0)
        hi = jnp.dot(g, w[i * w_in:(i + 1) * w_in],
                     preferred_element_type=jnp.float32)
        h = hi if h is None else h + hi
    hs = (h * dis).astype(jnp.bfloat16)
    for i in range(s_out):
        out_ref[0, i] = hs[:, i * w_out:(i + 1) * w_out]


def _tcmid_call(acc, dis, W, bias, w_in, s_in, w_out, s_out):
    d_in, d_out = w_in * s_in, w_out * s_out
    return pl.pallas_call(
        functools.partial(_tcmid_body, w_in, s_in, w_out, s_out),
        grid=(B, NB),
        in_specs=[
            pl.BlockSpec((1, s_in, RB, w_in), lambda b, n: (b, 0, n, 0)),
            pl.BlockSpec((1, RB, 1), lambda b, n: (b, n, 0)),
            pl.BlockSpec((d_in, d_out), lambda b, n: (0, 0)),
            pl.BlockSpec((d_in,), lambda b, n: (0,)),
        ],
        out_specs=pl.BlockSpec((1, s_out, RB, w_out),
                               lambda b, n: (b, 0, n, 0)),
        out_shape=jax.ShapeDtypeStruct((B, s_out, NP, w_out), jnp.bfloat16),
    )(acc, dis, W, bias)


def _tc4_body(acc_ref, dis_ref, b_ref, wfc_ref, bfc_ref, out_ref, pool_scr):
    n = pl.program_id(1)
    dis = dis_ref[0]
    bias = b_ref[...]
    w_in = D3 // 4
    rowid = lax.broadcasted_iota(jnp.int32, (RB, 1), 0) + n * RB
    m = (rowid < N).astype(jnp.float32)
    parts = []
    for i in range(4):
        g = jnp.maximum(acc_ref[0, i].astype(jnp.float32) * dis
                        + bias[i * w_in:(i + 1) * w_in][None, :], 0.0)
        parts.append(jnp.sum(g * m, axis=0))
    p = jnp.concatenate(parts)[None, :]

    @pl.when(n == 0)
    def _():
        pool_scr[...] = p

    @pl.when(n > 0)
    def _():
        pool_scr[...] = pool_scr[...] + p

    @pl.when(n == NB - 1)
    def _():
        pooled = pool_scr[...] * (1.0 / N)
        res = (jnp.dot(pooled, wfc_ref[...],
                       preferred_element_type=jnp.float32)
               + bfc_ref[...][None, :])
        out_ref[0] = jnp.broadcast_to(res, (8, D_OUT))


def _tc4_call(acc, dis, b3, Wfc, bfc):
    return pl.pallas_call(
        _tc4_body,
        grid=(B, NB),
        in_specs=[
            pl.BlockSpec((1, 4, RB, D3 // 4), lambda b, n: (b, 0, n, 0)),
            pl.BlockSpec((1, RB, 1), lambda b, n: (b, n, 0)),
            pl.BlockSpec((D3,), lambda b, n: (0,)),
            pl.BlockSpec((D3, D_OUT), lambda b, n: (0, 0)),
            pl.BlockSpec((D_OUT,), lambda b, n: (0,)),
        ],
        out_specs=pl.BlockSpec((1, 8, D_OUT), lambda b, n: (b, 0, 0)),
        out_shape=jax.ShapeDtypeStruct((B, 8, D_OUT), jnp.float32),
        scratch_shapes=[pltpu.VMEM((1, D3), jnp.float32)],
    )(acc, dis, b3, Wfc, bfc)


def kernel(x, edge_index, W1, b1, W2, b2, W3, b3, Wfc, bfc):
    src = edge_index[:, 0, :].astype(jnp.int32)
    dst = edge_index[:, 1, :].astype(jnp.int32)
    src_p = jnp.pad(src, ((0, 0), (0, EP - E)))
    dst_p = jnp.pad(dst, ((0, 0), (0, EP - E)), constant_values=TRASH)
    x_p = jnp.pad(x, ((0, 0), (0, NP - N), (0, 0)))

    cnt2 = _deg_call(dst_p).reshape(2, B, NP, 1)
    hs1, dis = _tc1_call(x_p, W1, cnt2)
    acc1 = _scatter1(hs1.reshape(B * 2 * NP, D1 // 2), src_p, dst_p)
    hs2 = _tcmid_call(acc1.reshape(B, 2, NP, D1 // 2), dis, W2, b1,
                      D1 // 2, 2, D2 // 2, 2)
    acc2 = _scatter2(hs2.reshape(B * 2 * NP, D2 // 2), src_p, dst_p)
    hs3 = _tcmid_call(acc2.reshape(B, 2, NP, D2 // 2), dis, W3, b2,
                      D2 // 2, 2, D3 // 4, 4)
    acc3 = _scatter3(hs3.reshape(B * 4 * NP, D3 // 4), src_p, dst_p)
    return _tc4_call(acc3.reshape(B, 4, NP, D3 // 4), dis, b3, Wfc, bfc)[:, 0, :]


# speedup vs baseline: 17.7664x; 1.4556x over previous
import functools

import jax
import jax.numpy as jnp
from jax import lax
from jax.experimental import pallas as pl
from jax.experimental.pallas import tpu as pltpu
from jax.experimental.pallas import tpu_sc as plsc

B, N, E = 4, 10000, 160000
D_IN, D1, D2, D3, D_OUT = 128, 64, 128, 256, 512

NP = 10240
TRASH = N
EP = 163840
K = 128
RT = NP // 16
EPT = EP // 16
EPT2 = EP // 32
RB = 1024
NB = NP // RB

_MESH = dict(core_axis_name="c", subcore_axis_name="s")
_SC_PARAMS = pltpu.CompilerParams(use_tc_tiling_on_sc=False)


G2 = 4
GK2 = G2 * K
NG2 = EPT2 // GK2


def _deg_call(dst_p):
    mesh = plsc.VectorSubcoreMesh(**_MESH)
    dstv = dst_p.reshape(B, 2, 16, NG2, G2, K)

    @functools.partial(
        pl.kernel,
        out_type=jax.ShapeDtypeStruct((2, B, NP), jnp.float32),
        mesh=mesh,
        compiler_params=_SC_PARAMS,
        scratch_types=[
            pltpu.VMEM_SHARED((NP,), jnp.float32),
            pltpu.VMEM((RT,), jnp.float32),
            pltpu.VMEM((K,), jnp.float32),
            pltpu.VMEM((G2, K), jnp.int32),
            pltpu.VMEM((G2, K), jnp.int32),
            pltpu.SemaphoreType.DMA, pltpu.SemaphoreType.DMA,
            pltpu.SemaphoreType.DMA, pltpu.SemaphoreType.DMA,
            pltpu.SemaphoreType.DMA,
        ],
    )
    def k(dst_hbm, cnt_hbm, deg_sh, zbuf, ones_v, didx0, didx1,
          isem0, isem1, ssem0, ssem1, wsem):
        c = lax.axis_index("c")
        s = lax.axis_index("s")
        didx = [didx0, didx1]
        isem = [isem0, isem1]
        ssem = [ssem0, ssem1]

        @pl.loop(0, RT, step=16)
        def _(i):
            zbuf[pl.ds(i, 16)] = jnp.zeros((16,), jnp.float32)

        @pl.loop(0, K, step=16)
        def _(i):
            ones_v[pl.ds(i, 16)] = jnp.ones((16,), jnp.float32)

        @pl.loop(0, B)
        def _(b):
            def load_idx(g, p):
                pltpu.async_copy(dst_hbm.at[b, c, s, g], didx[p], isem[p])

            def wait_idx(p):
                pltpu.make_async_copy(dst_hbm.at[b, c, s, 0], didx[p],
                                      isem[p]).wait()

            def start_scatters(p):
                for j in range(G2):
                    pltpu.async_copy(ones_v, deg_sh.at[didx[p].at[j]],
                                     ssem[p], add=True)

            def wait_scatters(p):
                for j in range(G2):
                    pltpu.make_async_copy(ones_v, deg_sh.at[didx[p].at[j]],
                                          ssem[p]).wait()

            init = pltpu.async_copy(zbuf, deg_sh.at[pl.ds(s * RT, RT)], wsem)
            load_idx(0, 0)
            load_idx(1, 1)
            init.wait()
            plsc.subcore_barrier()

            @pl.loop(0, (NG2 - 2) // 2)
            def _(t):
                g = 2 * t
                wait_idx(0)
                start_scatters(0)
                wait_idx(1)
                start_scatters(1)
                wait_scatters(0)
                load_idx(g + 2, 0)
                wait_scatters(1)
                load_idx(g + 3, 1)

            wait_idx(0)
            start_scatters(0)
            wait_idx(1)
            start_scatters(1)
            wait_scatters(0)
            wait_scatters(1)
            plsc.subcore_barrier()
            pltpu.sync_copy(deg_sh.at[pl.ds(s * RT, RT)],
                            cnt_hbm.at[c, b, pl.ds(s * RT, RT)])

    return k(dstv)


def _make_scatter(W, npass):
    GK = 1024
    NG = EPT // GK
    NS = 2 * npass
    mesh = plsc.VectorSubcoreMesh(**_MESH)

    @functools.partial(
        pl.kernel,
        out_type=jax.ShapeDtypeStruct((B * NS, NP, W), jnp.bfloat16),
        mesh=mesh,
        compiler_params=_SC_PARAMS,
        scratch_types=[
            pltpu.VMEM_SHARED((NP, W), jnp.bfloat16),
            pltpu.VMEM((GK,), jnp.int32), pltpu.VMEM((GK,), jnp.int32),
            pltpu.VMEM((GK,), jnp.int32), pltpu.VMEM((GK,), jnp.int32),
            pltpu.VMEM((GK, W), jnp.bfloat16),
            pltpu.VMEM((GK, W), jnp.bfloat16),
            pltpu.SemaphoreType.DMA, pltpu.SemaphoreType.DMA,
            pltpu.SemaphoreType.DMA, pltpu.SemaphoreType.DMA,
            pltpu.SemaphoreType.DMA, pltpu.SemaphoreType.DMA,
            pltpu.SemaphoreType.DMA,
        ],
    )
    def k(hs_hbm, srcv, dstv, out_hbm, acc_sh,
          sidx0, sidx1, didx0, didx1, rows0, rows1,
          isem0, isem1, gsem0, gsem1, ssem0, ssem1, wsem):
        c = lax.axis_index("c")
        s = lax.axis_index("s")
        sidx = [sidx0, sidx1]
        didx = [didx0, didx1]
        rows = [rows0, rows1]
        isem = [isem0, isem1]
        gsem = [gsem0, gsem1]
        ssem = [ssem0, ssem1]

        @pl.loop(0, B * npass)
        def _(u):
            b = u // npass
            q = u - b * npass
            bcq = (b * 2 + c) * npass + q

            def load_idx(g, p):
                pltpu.async_copy(srcv.at[b, s, g], sidx[p], isem[p])
                pltpu.async_copy(dstv.at[b, s, g], didx[p], isem[p])

            def start_gathers(p):
                pltpu.make_async_copy(srcv.at[b, s, 0], sidx[p],
                                      isem[p]).wait()
                pltpu.make_async_copy(dstv.at[b, s, 0], didx[p],
                                      isem[p]).wait()
                pltpu.async_copy(hs_hbm.at[bcq].at[sidx[p]], rows[p],
                                 gsem[p])

            def wait_gathers(p):
                pltpu.make_async_copy(hs_hbm.at[bcq].at[sidx[p]], rows[p],
                                      gsem[p]).wait()

            def start_scatters(p):
                pltpu.async_copy(rows[p], acc_sh.at[didx[p]], ssem[p],
                                 add=True)

            def wait_scatters(p):
                pltpu.make_async_copy(rows[p], acc_sh.at[didx[p]],
                                      ssem[p]).wait()

            init = pltpu.async_copy(hs_hbm.at[bcq, pl.ds(s * RT, RT)],
                                    acc_sh.at[pl.ds(s * RT, RT)], wsem)
            load_idx(0, 0)
            load_idx(1, 1)
            start_gathers(0)
            init.wait()
            plsc.subcore_barrier()

            @pl.loop(0, (NG - 2) // 2)
            def _(t):
                g = 2 * t
                wait_gathers(0)
                start_scatters(0)
                start_gathers(1)
                wait_scatters(0)
                load_idx(g + 2, 0)
                wait_gathers(1)
                start_scatters(1)
                start_gathers(0)
                wait_scatters(1)
                load_idx(g + 3, 1)

            wait_gathers(0)
            start_scatters(0)
            start_gathers(1)
            wait_scatters(0)
            wait_gathers(1)
            start_scatters(1)
            wait_scatters(1)
            plsc.subcore_barrier()
            pltpu.sync_copy(acc_sh.at[pl.ds(s * RT, RT)],
                            out_hbm.at[bcq, pl.ds(s * RT, RT)])

    def call(hs_flat, src_p, dst_p):
        return k(hs_flat.reshape(B * NS, NP, W),
                 src_p.reshape(B, 16, NG, GK),
                 dst_p.reshape(B, 16, NG, GK))

    return call


_scatter_w32 = _make_scatter(32, 1)
_scatter_w64 = _make_scatter(64, 1)


def _tc1_body(x_ref, w_ref, cnt_ref, hs_ref, dis_ref):
    deg = cnt_ref[0, 0] + cnt_ref[1, 0] + 1.0
    dis = lax.rsqrt(deg)
    h = jnp.dot(x_ref[0], w_ref[...], preferred_element_type=jnp.float32)
    hs = (h * dis).astype(jnp.bfloat16)
    w_out = D1 // 2
    for i in range(2):
        hs_ref[0, i] = hs[:, i * w_out:(i + 1) * w_out]
    dis_ref[0] = dis


def _tc1_call(x_p, W1, cnt2):
    return pl.pallas_call(
        _tc1_body,
        grid=(B, NB),
        in_specs=[
            pl.BlockSpec((1, RB, D_IN), lambda b, n: (b, n, 0)),
            pl.BlockSpec((D_IN, D1), lambda b, n: (0, 0)),
            pl.BlockSpec((2, 1, RB, 1), lambda b, n: (0, b, n, 0)),
        ],
        out_specs=[
            pl.BlockSpec((1, 2, RB, D1 // 2), lambda b, n: (b, 0, n, 0)),
            pl.BlockSpec((1, RB, 1), lambda b, n: (b, n, 0)),
        ],
        out_shape=[
            jax.ShapeDtypeStruct((B, 2, NP, D1 // 2), jnp.bfloat16),
            jax.ShapeDtypeStruct((B, NP, 1), jnp.float32),
        ],
    )(x_p, W1, cnt2)


def _tc2_body(acc_ref, dis_ref, b_ref, out_ref):
    dis = dis_ref[0]
    bias = b_ref[...]
    for i in range(2):
        g = jnp.maximum(acc_ref[0, i].astype(jnp.float32) * dis
                        + bias[i * 32:(i + 1) * 32][None, :], 0.0)
        out_ref[0, i] = (g * dis).astype(jnp.bfloat16)


def _tc2_call(acc, dis, b1):
    return pl.pallas_call(
        _tc2_body,
        grid=(B, NB),
        in_specs=[
            pl.BlockSpec((1, 2, RB, 32), lambda b, n: (b, 0, n, 0)),
            pl.BlockSpec((1, RB, 1), lambda b, n: (b, n, 0)),
            pl.BlockSpec((D1,), lambda b, n: (0,)),
        ],
        out_specs=pl.BlockSpec((1, 2, RB, 32), lambda b, n: (b, 0, n, 0)),
        out_shape=jax.ShapeDtypeStruct((B, 2, NP, 32), jnp.bfloat16),
    )(acc, dis, b1)


def _tc3_body(acc_ref, dis_ref, w_ref, b_ref, out_ref):
    dis = dis_ref[0]
    bias = b_ref[...]
    w = w_ref[...]
    h = None
    for i in range(2):
        hi = jnp.dot(acc_ref[0, i].astype(jnp.float32),
                     w[i * 32:(i + 1) * 32],
                     preferred_element_type=jnp.float32)
        h = hi if h is None else h + hi
    g = jnp.maximum(h * dis + bias[None, :], 0.0)
    z = (g * dis).astype(jnp.bfloat16)
    for i in range(2):
        out_ref[0, i] = z[:, i * 64:(i + 1) * 64]


def _tc3_call(acc, dis, W2, b2):
    return pl.pallas_call(
        _tc3_body,
        grid=(B, NB),
        in_specs=[
            pl.BlockSpec((1, 2, RB, 32), lambda b, n: (b, 0, n, 0)),
            pl.BlockSpec((1, RB, 1), lambda b, n: (b, n, 0)),
            pl.BlockSpec((D1, D2), lambda b, n: (0, 0)),
            pl.BlockSpec((D2,), lambda b, n: (0,)),
        ],
        out_specs=pl.BlockSpec((1, 2, RB, 64), lambda b, n: (b, 0, n, 0)),
        out_shape=jax.ShapeDtypeStruct((B, 2, NP, 64), jnp.bfloat16),
    )(acc, dis, W2, b2)


def _tc4_body(acc_ref, dis_ref, w3_ref, b_ref, wfc_ref, bfc_ref, out_ref,
              pool_scr):
    n = pl.program_id(1)
    dis = dis_ref[0]
    bias = b_ref[...]
    w3 = w3_ref[...]
    h = None
    for i in range(2):
        hi = jnp.dot(acc_ref[0, i].astype(jnp.float32),
                     w3[i * 64:(i + 1) * 64],
                     preferred_element_type=jnp.float32)
        h = hi if h is None else h + hi
    g = jnp.maximum(h * dis + bias[None, :], 0.0)
    rowid = lax.broadcasted_iota(jnp.int32, (RB, 1), 0) + n * RB
    m = (rowid < N).astype(jnp.float32)
    p = jnp.sum(g * m, axis=0)[None, :]

    @pl.when(n == 0)
    def _():
        pool_scr[...] = p

    @pl.when(n > 0)
    def _():
        pool_scr[...] = pool_scr[...] + p

    @pl.when(n == NB - 1)
    def _():
        pooled = pool_scr[...] * (1.0 / N)
        res = (jnp.dot(pooled, wfc_ref[...],
                       preferred_element_type=jnp.float32)
               + bfc_ref[...][None, :])
        out_ref[0] = jnp.broadcast_to(res, (8, D_OUT))


def _tc4_call(acc, dis, W3, b3, Wfc, bfc):
    return pl.pallas_call(
        _tc4_body,
        grid=(B, NB),
        in_specs=[
            pl.BlockSpec((1, 2, RB, 64), lambda b, n: (b, 0, n, 0)),
            pl.BlockSpec((1, RB, 1), lambda b, n: (b, n, 0)),
            pl.BlockSpec((D2, D3), lambda b, n: (0, 0)),
            pl.BlockSpec((D3,), lambda b, n: (0,)),
            pl.BlockSpec((D3, D_OUT), lambda b, n: (0, 0)),
            pl.BlockSpec((D_OUT,), lambda b, n: (0,)),
        ],
        out_specs=pl.BlockSpec((1, 8, D_OUT), lambda b, n: (b, 0, 0)),
        out_shape=jax.ShapeDtypeStruct((B, 8, D_OUT), jnp.float32),
        scratch_shapes=[pltpu.VMEM((1, D3), jnp.float32)],
    )(acc, dis, W3, b3, Wfc, bfc)


def kernel(x, edge_index, W1, b1, W2, b2, W3, b3, Wfc, bfc):
    src = edge_index[:, 0, :].astype(jnp.int32)
    dst = edge_index[:, 1, :].astype(jnp.int32)
    src_p = jnp.pad(src, ((0, 0), (0, EP - E)))
    dst_p = jnp.pad(dst, ((0, 0), (0, EP - E)), constant_values=TRASH)
    x_p = jnp.pad(x, ((0, 0), (0, NP - N), (0, 0)))

    cnt2 = _deg_call(dst_p).reshape(2, B, NP, 1)
    hs1, dis = _tc1_call(x_p, W1, cnt2)
    agg1 = _scatter_w32(hs1.reshape(B * 2 * NP, 32), src_p, dst_p)
    z2 = _tc2_call(agg1.reshape(B, 2, NP, 32), dis, b1)
    agg2 = _scatter_w32(z2.reshape(B * 2 * NP, 32), src_p, dst_p)
    z3 = _tc3_call(agg2.reshape(B, 2, NP, 32), dis, W2, b2)
    agg3 = _scatter_w64(z3.reshape(B * 2 * NP, 64), src_p, dst_p)
    return _tc4_call(agg3.reshape(B, 2, NP, 64), dis, W3, b3, Wfc,
                     bfc)[:, 0, :]


# speedup vs baseline: 21.9203x; 1.2338x over previous
import functools

import jax
import jax.numpy as jnp
from jax import lax
from jax.experimental import pallas as pl
from jax.experimental.pallas import tpu as pltpu
from jax.experimental.pallas import tpu_sc as plsc

B, N, E = 4, 10000, 160000
D_IN, D1, D2, D3, D_OUT = 128, 64, 128, 256, 512

NP = 10240
TRASH = N
EP = 163840
K = 128
RT = NP // 16
EPT = EP // 16
EPT2 = EP // 32
RB = 1024
NB = NP // RB

_MESH = dict(core_axis_name="c", subcore_axis_name="s")
_SC_PARAMS = pltpu.CompilerParams(use_tc_tiling_on_sc=False)


G2 = 4
GK2 = G2 * K
NG2 = EPT2 // GK2


def _deg_call(dst_p):
    mesh = plsc.VectorSubcoreMesh(**_MESH)
    dstv = dst_p.reshape(B, 2, 16, NG2, G2, K)

    @functools.partial(
        pl.kernel,
        out_type=jax.ShapeDtypeStruct((2, B, NP), jnp.float32),
        mesh=mesh,
        compiler_params=_SC_PARAMS,
        scratch_types=[
            pltpu.VMEM_SHARED((NP,), jnp.float32),
            pltpu.VMEM((RT,), jnp.float32),
            pltpu.VMEM((K,), jnp.float32),
            pltpu.VMEM((G2, K), jnp.int32),
            pltpu.VMEM((G2, K), jnp.int32),
            pltpu.SemaphoreType.DMA, pltpu.SemaphoreType.DMA,
            pltpu.SemaphoreType.DMA, pltpu.SemaphoreType.DMA,
            pltpu.SemaphoreType.DMA,
        ],
    )
    def k(dst_hbm, cnt_hbm, deg_sh, zbuf, ones_v, didx0, didx1,
          isem0, isem1, ssem0, ssem1, wsem):
        c = lax.axis_index("c")
        s = lax.axis_index("s")
        didx = [didx0, didx1]
        isem = [isem0, isem1]
        ssem = [ssem0, ssem1]

        @pl.loop(0, RT, step=16)
        def _(i):
            zbuf[pl.ds(i, 16)] = jnp.zeros((16,), jnp.float32)

        @pl.loop(0, K, step=16)
        def _(i):
            ones_v[pl.ds(i, 16)] = jnp.ones((16,), jnp.float32)

        @pl.loop(0, B)
        def _(b):
            def load_idx(g, p):
                pltpu.async_copy(dst_hbm.at[b, c, s, g], didx[p], isem[p])

            def wait_idx(p):
                pltpu.make_async_copy(dst_hbm.at[b, c, s, 0], didx[p],
                                      isem[p]).wait()

            def start_scatters(p):
                for j in range(G2):
                    pltpu.async_copy(ones_v, deg_sh.at[didx[p].at[j]],
                                     ssem[p], add=True)

            def wait_scatters(p):
                for j in range(G2):
                    pltpu.make_async_copy(ones_v, deg_sh.at[didx[p].at[j]],
                                          ssem[p]).wait()

            init = pltpu.async_copy(zbuf, deg_sh.at[pl.ds(s * RT, RT)], wsem)
            load_idx(0, 0)
            load_idx(1, 1)
            init.wait()
            plsc.subcore_barrier()

            @pl.loop(0, (NG2 - 2) // 2)
            def _(t):
                g = 2 * t
                wait_idx(0)
                start_scatters(0)
                wait_idx(1)
                start_scatters(1)
                wait_scatters(0)
                load_idx(g + 2, 0)
                wait_scatters(1)
                load_idx(g + 3, 1)

            wait_idx(0)
            start_scatters(0)
            wait_idx(1)
            start_scatters(1)
            wait_scatters(0)
            wait_scatters(1)
            plsc.subcore_barrier()
            pltpu.sync_copy(deg_sh.at[pl.ds(s * RT, RT)],
                            cnt_hbm.at[c, b, pl.ds(s * RT, RT)])

    return k(dstv)


def _make_scatter(W, bv, hsb):
    GK = 1024
    NG = EPT // GK
    mesh = plsc.VectorSubcoreMesh(**_MESH)

    @functools.partial(
        pl.kernel,
        out_type=jax.ShapeDtypeStruct((2, NP, W), jnp.bfloat16),
        mesh=mesh,
        compiler_params=_SC_PARAMS,
        scratch_types=[
            pltpu.VMEM_SHARED((NP, W), jnp.bfloat16),
            pltpu.VMEM((GK,), jnp.int32), pltpu.VMEM((GK,), jnp.int32),
            pltpu.VMEM((GK,), jnp.int32), pltpu.VMEM((GK,), jnp.int32),
            pltpu.VMEM((GK, W), jnp.bfloat16),
            pltpu.VMEM((GK, W), jnp.bfloat16),
            pltpu.SemaphoreType.DMA, pltpu.SemaphoreType.DMA,
            pltpu.SemaphoreType.DMA, pltpu.SemaphoreType.DMA,
            pltpu.SemaphoreType.DMA, pltpu.SemaphoreType.DMA,
            pltpu.SemaphoreType.DMA,
        ],
    )
    def k(hs_hbm, srcv, dstv, out_hbm, acc_sh,
          sidx0, sidx1, didx0, didx1, rows0, rows1,
          isem0, isem1, gsem0, gsem1, ssem0, ssem1, wsem):
        c = lax.axis_index("c")
        s = lax.axis_index("s")
        sidx = [sidx0, sidx1]
        didx = [didx0, didx1]
        rows = [rows0, rows1]
        isem = [isem0, isem1]
        gsem = [gsem0, gsem1]
        ssem = [ssem0, ssem1]
        bc = hsb + c

        def load_idx(g, p):
            pltpu.async_copy(srcv.at[bv, s, g], sidx[p], isem[p])
            pltpu.async_copy(dstv.at[bv, s, g], didx[p], isem[p])

        def start_gathers(p):
            pltpu.make_async_copy(srcv.at[bv, s, 0], sidx[p],
                                  isem[p]).wait()
            pltpu.make_async_copy(dstv.at[bv, s, 0], didx[p],
                                  isem[p]).wait()
            pltpu.async_copy(hs_hbm.at[bc].at[sidx[p]], rows[p], gsem[p])

        def wait_gathers(p):
            pltpu.make_async_copy(hs_hbm.at[bc].at[sidx[p]], rows[p],
                                  gsem[p]).wait()

        def start_scatters(p):
            pltpu.async_copy(rows[p], acc_sh.at[didx[p]], ssem[p],
                             add=True)

        def wait_scatters(p):
            pltpu.make_async_copy(rows[p], acc_sh.at[didx[p]],
                                  ssem[p]).wait()

        init = pltpu.async_copy(hs_hbm.at[bc, pl.ds(s * RT, RT)],
                                acc_sh.at[pl.ds(s * RT, RT)], wsem)
        load_idx(0, 0)
        load_idx(1, 1)
        start_gathers(0)
        init.wait()
        plsc.subcore_barrier()

        @pl.loop(0, (NG - 2) // 2)
        def _(t):
            g = 2 * t
            wait_gathers(0)
            start_scatters(0)
            start_gathers(1)
            wait_scatters(0)
            load_idx(g + 2, 0)
            wait_gathers(1)
            start_scatters(1)
            start_gathers(0)
            wait_scatters(1)
            load_idx(g + 3, 1)

        wait_gathers(0)
        start_scatters(0)
        start_gathers(1)
        wait_scatters(0)
        wait_gathers(1)
        start_scatters(1)
        wait_scatters(1)
        plsc.subcore_barrier()
        pltpu.sync_copy(acc_sh.at[pl.ds(s * RT, RT)],
                        out_hbm.at[c, pl.ds(s * RT, RT)])

    def call(hs_flat, srcv, dstv):
        return k(hs_flat.reshape(-1, NP, W), srcv, dstv)

    return call


_scatter_l1 = [_make_scatter(32, b, 2 * b) for b in range(B)]
_scatter_l2 = [_make_scatter(32, b, 0) for b in range(B)]
_scatter_l3 = [_make_scatter(64, b, 0) for b in range(B)]


def _tc1_body(x_ref, w_ref, cnt_ref, hs_ref, dis_ref):
    deg = cnt_ref[0, 0] + cnt_ref[1, 0] + 1.0
    dis = lax.rsqrt(deg)
    h = jnp.dot(x_ref[0], w_ref[...], preferred_element_type=jnp.float32)
    hs = (h * dis).astype(jnp.bfloat16)
    w_out = D1 // 2
    for i in range(2):
        hs_ref[0, i] = hs[:, i * w_out:(i + 1) * w_out]
    dis_ref[0] = dis


def _tc1_call(x_p, W1, cnt2):
    return pl.pallas_call(
        _tc1_body,
        grid=(B, NB),
        in_specs=[
            pl.BlockSpec((1, RB, D_IN), lambda b, n: (b, n, 0)),
            pl.BlockSpec((D_IN, D1), lambda b, n: (0, 0)),
            pl.BlockSpec((2, 1, RB, 1), lambda b, n: (0, b, n, 0)),
        ],
        out_specs=[
            pl.BlockSpec((1, 2, RB, D1 // 2), lambda b, n: (b, 0, n, 0)),
            pl.BlockSpec((1, RB, 1), lambda b, n: (b, n, 0)),
        ],
        out_shape=[
            jax.ShapeDtypeStruct((B, 2, NP, D1 // 2), jnp.bfloat16),
            jax.ShapeDtypeStruct((B, NP, 1), jnp.float32),
        ],
    )(x_p, W1, cnt2)


def _tc2_body(acc_ref, dis_ref, b_ref, out_ref):
    dis = dis_ref[0]
    bias = b_ref[...]
    for i in range(2):
        g = jnp.maximum(acc_ref[i].astype(jnp.float32) * dis
                        + bias[i * 32:(i + 1) * 32][None, :], 0.0)
        out_ref[i] = (g * dis).astype(jnp.bfloat16)


def _tc2_call(bv, acc, dis, b1):
    return pl.pallas_call(
        _tc2_body,
        grid=(NB,),
        in_specs=[
            pl.BlockSpec((2, RB, 32), lambda n: (0, n, 0)),
            pl.BlockSpec((1, RB, 1), lambda n: (bv, n, 0)),
            pl.BlockSpec((D1,), lambda n: (0,)),
        ],
        out_specs=pl.BlockSpec((2, RB, 32), lambda n: (0, n, 0)),
        out_shape=jax.ShapeDtypeStruct((2, NP, 32), jnp.bfloat16),
    )(acc, dis, b1)


def _tc3_body(acc_ref, dis_ref, w_ref, b_ref, out_ref):
    dis = dis_ref[0]
    bias = b_ref[...]
    w = w_ref[...]
    h = None
    for i in range(2):
        hi = jnp.dot(acc_ref[i].astype(jnp.float32),
                     w[i * 32:(i + 1) * 32],
                     preferred_element_type=jnp.float32)
        h = hi if h is None else h + hi
    g = jnp.maximum(h * dis + bias[None, :], 0.0)
    z = (g * dis).astype(jnp.bfloat16)
    for i in range(2):
        out_ref[i] = z[:, i * 64:(i + 1) * 64]


def _tc3_call(bv, acc, dis, W2, b2):
    return pl.pallas_call(
        _tc3_body,
        grid=(NB,),
        in_specs=[
            pl.BlockSpec((2, RB, 32), lambda n: (0, n, 0)),
            pl.BlockSpec((1, RB, 1), lambda n: (bv, n, 0)),
            pl.BlockSpec((D1, D2), lambda n: (0, 0)),
            pl.BlockSpec((D2,), lambda n: (0,)),
        ],
        out_specs=pl.BlockSpec((2, RB, 64), lambda n: (0, n, 0)),
        out_shape=jax.ShapeDtypeStruct((2, NP, 64), jnp.bfloat16),
    )(acc, dis, W2, b2)


def _tc4_body(acc_ref, dis_ref, w3_ref, b_ref, wfc_ref, bfc_ref, out_ref,
              pool_scr):
    n = pl.program_id(0)
    dis = dis_ref[0]
    bias = b_ref[...]
    w3 = w3_ref[...]
    h = None
    for i in range(2):
        hi = jnp.dot(acc_ref[i].astype(jnp.float32),
                     w3[i * 64:(i + 1) * 64],
                     preferred_element_type=jnp.float32)
        h = hi if h is None else h + hi
    g = jnp.maximum(h * dis + bias[None, :], 0.0)
    rowid = lax.broadcasted_iota(jnp.int32, (RB, 1), 0) + n * RB
    m = (rowid < N).astype(jnp.float32)
    p = jnp.sum(g * m, axis=0)[None, :]

    @pl.when(n == 0)
    def _():
        pool_scr[...] = p

    @pl.when(n > 0)
    def _():
        pool_scr[...] = pool_scr[...] + p

    @pl.when(n == NB - 1)
    def _():
        pooled = pool_scr[...] * (1.0 / N)
        res = (jnp.dot(pooled, wfc_ref[...],
                       preferred_element_type=jnp.float32)
               + bfc_ref[...][None, :])
        out_ref[...] = jnp.broadcast_to(res, (8, D_OUT))


def _tc4_call(bv, acc, dis, W3, b3, Wfc, bfc):
    return pl.pallas_call(
        _tc4_body,
        grid=(NB,),
        in_specs=[
            pl.BlockSpec((2, RB, 64), lambda n: (0, n, 0)),
            pl.BlockSpec((1, RB, 1), lambda n: (bv, n, 0)),
            pl.BlockSpec((D2, D3), lambda n: (0, 0)),
            pl.BlockSpec((D3,), lambda n: (0,)),
            pl.BlockSpec((D3, D_OUT), lambda n: (0, 0)),
            pl.BlockSpec((D_OUT,), lambda n: (0,)),
        ],
        out_specs=pl.BlockSpec((8, D_OUT), lambda n: (0, 0)),
        out_shape=jax.ShapeDtypeStruct((8, D_OUT), jnp.float32),
        scratch_shapes=[pltpu.VMEM((1, D3), jnp.float32)],
    )(acc, dis, W3, b3, Wfc, bfc)


def kernel(x, edge_index, W1, b1, W2, b2, W3, b3, Wfc, bfc):
    src = edge_index[:, 0, :].astype(jnp.int32)
    dst = edge_index[:, 1, :].astype(jnp.int32)
    src_p = jnp.pad(src, ((0, 0), (0, EP - E)))
    dst_p = jnp.pad(dst, ((0, 0), (0, EP - E)), constant_values=TRASH)
    x_p = jnp.pad(x, ((0, 0), (0, NP - N), (0, 0)))
    srcv = src_p.reshape(B, 16, EPT // 1024, 1024)
    dstv = dst_p.reshape(B, 16, EPT // 1024, 1024)

    cnt2 = _deg_call(dst_p).reshape(2, B, NP, 1)
    hs1, dis = _tc1_call(x_p, W1, cnt2)
    hs1f = hs1.reshape(B * 2 * NP, 32)
    outs = []
    for b in range(B):
        agg1 = _scatter_l1[b](hs1f, srcv, dstv)
        z2 = _tc2_call(b, agg1, dis, b1)
        agg2 = _scatter_l2[b](z2.reshape(2 * NP, 32), srcv, dstv)
        z3 = _tc3_call(b, agg2, dis, W2, b2)
        agg3 = _scatter_l3[b](z3.reshape(2 * NP, 64), srcv, dstv)
        outs.append(_tc4_call(b, agg3, dis, W3, b3, Wfc, bfc)[0:1])
    return jnp.concatenate(outs, axis=0)
